# Initial kernel scaffold; baseline (speedup 1.0000x reference)
import jax
import jax.numpy as jnp
from jax.experimental import pallas as pl


def kernel(hidden_states, attn_wq, attn_wk, attn_wv, attn_wo, idx_wq, idx_wk, idx_w, norm1_w, norm2_w, gate_w, up_w, down_w):
    raise NotImplementedError("write your pallas kernel here")


# speedup vs baseline: 5.9304x; 5.9304x over previous
import math

import numpy as np
import jax
import jax.numpy as jnp
from jax.experimental import pallas as pl
from jax.experimental.pallas import tpu as pltpu

_S, _D = 2048, 2048
_H, _DH = 16, 128
_HI, _DI = 4, 64
_K = 512
_FF = 5632
_EPS = 1e-6

_NCAT = 6656

_BM, _BN = 512, 512
_BQR = 256
_BQ, _BK = 256, 512

_INTERPRET = False


def _rope_cache_np():
    pos = np.arange(_S)
    inv = 1.0 / (10000.0 ** (np.arange(0, _DH, 2) / _DH))
    freqs = np.outer(pos, inv)
    emb = np.concatenate([freqs, freqs], axis=-1)
    return np.cos(emb).astype(np.float32), np.sin(emb).astype(np.float32)


_COS_NP, _SIN_NP = _rope_cache_np()


def _rot_half(x):
    half = x.shape[-1] // 2
    return jnp.concatenate([-x[:, half:], x[:, :half]], axis=1)


def _norm_matmul_rope_kernel(x_ref, nw_ref, w_ref, cos_ref, sin_ref, o_ref, xn_ref):
    j = pl.program_id(1)

    @pl.when(j == 0)
    def _():
        x = x_ref[...]
        xn_ref[...] = x * jax.lax.rsqrt(
            jnp.mean(x * x, axis=-1, keepdims=True) + _EPS) * nw_ref[...]

    y = jnp.dot(xn_ref[...], w_ref[...], preferred_element_type=jnp.float32)

    def rope(t):
        c, s = cos_ref[...], sin_ref[...]
        parts = []
        for hh in range(_BN // _DH):
            u = t[:, hh * _DH:(hh + 1) * _DH]
            parts.append(u * c + _rot_half(u) * s)
        return jnp.concatenate(parts, axis=1)

    @pl.when(j < 4)
    def _():
        o_ref[...] = rope(y) * (1.0 / math.sqrt(_DH))

    @pl.when((j >= 4) & (j < 8))
    def _():
        o_ref[...] = rope(y)

    @pl.when(j >= 8)
    def _():
        o_ref[...] = y


def _select_kernel(qi_ref, ki_ref, mask_ref):
    i = pl.program_id(0)
    qi = qi_ref[...]
    ki = ki_ref[:, :_DI]
    isc = None
    for h in range(_HI):
        part = jax.lax.dot_general(
            qi[:, h * _DI:(h + 1) * _DI], ki, (((1,), (1,)), ((), ())),
            preferred_element_type=jnp.float32,
            precision=jax.lax.Precision.HIGHEST)
        part = jnp.maximum(part, 0.0)
        isc = part if isc is None else isc + part

    rows = i * _BQR + jax.lax.broadcasted_iota(jnp.int32, (_BQR, _S), 0)
    cols = jax.lax.broadcasted_iota(jnp.int32, (_BQR, _S), 1)
    causal = cols <= rows
    key = jnp.where(causal, jax.lax.bitcast_convert_type(isc, jnp.int32),
                    jnp.int32(-1))

    c0 = jnp.sum((key >= 0).astype(jnp.int32), axis=1)
    has_k = c0 >= _K
    lo = jnp.where(has_k, 0, -1)
    hi = jnp.where(has_k, jnp.max(key, axis=1), -1)

    def bs_body(_, carry):
        lo, hi = carry
        mid = lo + (hi - lo + 1) // 2
        c = jnp.sum((key >= mid[:, None]).astype(jnp.int32), axis=1)
        ge = c >= _K
        return jnp.where(ge, mid, lo), jnp.where(ge, hi, mid - 1)

    lo, hi = jax.lax.fori_loop(0, 31, bs_body, (lo, hi))
    tau = lo

    m = jnp.sum((key > tau[:, None]).astype(jnp.int32), axis=1)
    t = _K - m
    tie = key == tau[:, None]

    def bs2_body(_, carry):
        lo2, hi2 = carry
        mid = (lo2 + hi2) // 2
        c = jnp.sum((tie & (cols <= mid[:, None])).astype(jnp.int32), axis=1)
        ge = c >= t
        return jnp.where(ge, lo2, mid + 1), jnp.where(ge, mid, hi2)

    lo2, hi2 = jax.lax.fori_loop(
        0, 11, bs2_body,
        (jnp.zeros((_BQR,), jnp.int32), jnp.full((_BQR,), _S - 1, jnp.int32)))
    jcut = lo2

    allow = causal & ((key > tau[:, None]) |
                      (tie & (cols <= jcut[:, None])))
    mask_ref[...] = allow.astype(jnp.int8)


def _attn_kernel(q_ref, k_ref, v_ref, mask_ref, o_ref):
    i = pl.program_id(1)
    q = q_ref[...]
    nb = (i * _BQ + _BQ + _BK - 1) // _BK

    def body(b, carry):
        m, l, acc = carry
        j0 = b * _BK
        kb = k_ref[pl.ds(j0, _BK), :]
        s = jax.lax.dot_general(q, kb, (((1,), (1,)), ((), ())),
                                preferred_element_type=jnp.float32)
        allow = mask_ref[:, pl.ds(j0, _BK)] != 0
        s = jnp.where(allow, s, -1e30)
        mn = jnp.maximum(m, jnp.max(s, axis=1))
        p = jnp.where(allow, jnp.exp(s - mn[:, None]), 0.0)
        alpha = jnp.exp(m - mn)
        l = l * alpha + jnp.sum(p, axis=1)
        acc = acc * alpha[:, None] + jnp.dot(
            p, v_ref[pl.ds(j0, _BK), :], preferred_element_type=jnp.float32)
        return mn, l, acc

    m0 = jnp.full((_BQ,), -1e30, jnp.float32)
    l0 = jnp.zeros((_BQ,), jnp.float32)
    a0 = jnp.zeros((_BQ, _DH), jnp.float32)
    m, l, acc = jax.lax.fori_loop(0, nb, body, (m0, l0, a0))
    o_ref[...] = acc / l[:, None]


def _matmul_res_kernel(x_ref, w_ref, r_ref, o_ref):
    o_ref[...] = r_ref[...] + jnp.dot(x_ref[...], w_ref[...],
                                      preferred_element_type=jnp.float32)


def _mlp_up_kernel(x_ref, nw_ref, g_ref, u_ref, o_ref, xn_ref):
    @pl.when(pl.program_id(1) == 0)
    def _():
        x = x_ref[...]
        xn_ref[...] = x * jax.lax.rsqrt(
            jnp.mean(x * x, axis=-1, keepdims=True) + _EPS) * nw_ref[...]

    xn = xn_ref[...]
    g = jnp.dot(xn, g_ref[...], preferred_element_type=jnp.float32)
    u = jnp.dot(xn, u_ref[...], preferred_element_type=jnp.float32)
    o_ref[...] = g * jax.lax.logistic(g) * u


def kernel(hidden_states, attn_wq, attn_wk, attn_wv, attn_wo, idx_wq, idx_wk,
           idx_w, norm1_w, norm2_w, gate_w, up_w, down_w):
    x = hidden_states[0]
    wqi = (idx_wq.reshape(_D, _HI, _DI) * idx_w[None, :, None]).reshape(
        _D, _HI * _DI)
    wcat = jnp.concatenate(
        [attn_wq, attn_wk, attn_wv, wqi, idx_wk,
         jnp.zeros((_D, _NCAT - 3 * _D - _HI * _DI - _DI), jnp.float32)],
        axis=1)
    nw1 = norm1_w.reshape(1, _D)
    nw2 = norm2_w.reshape(1, _D)
    cos = jnp.asarray(_COS_NP)
    sin = jnp.asarray(_SIN_NP)

    y = pl.pallas_call(
        _norm_matmul_rope_kernel,
        grid=(_S // _BM, _NCAT // _BN),
        in_specs=[
            pl.BlockSpec((_BM, _D), lambda i, j: (i, 0)),
            pl.BlockSpec((1, _D), lambda i, j: (0, 0)),
            pl.BlockSpec((_D, _BN), lambda i, j: (0, j)),
            pl.BlockSpec((_BM, _DH), lambda i, j: (i, 0)),
            pl.BlockSpec((_BM, _DH), lambda i, j: (i, 0)),
        ],
        out_specs=pl.BlockSpec((_BM, _BN), lambda i, j: (i, j)),
        out_shape=jax.ShapeDtypeStruct((_S, _NCAT), jnp.float32),
        scratch_shapes=[pltpu.VMEM((_BM, _D), jnp.float32)],
        interpret=_INTERPRET,
    )(x, nw1, wcat, cos, sin)

    allow = pl.pallas_call(
        _select_kernel,
        grid=(_S // _BQR,),
        in_specs=[
            pl.BlockSpec((_BQR, _HI * _DI), lambda i: (i, 24)),
            pl.BlockSpec((_S, 2 * _DI), lambda i: (0, 50)),
        ],
        out_specs=pl.BlockSpec((_BQR, _S), lambda i: (i, 0)),
        out_shape=jax.ShapeDtypeStruct((_S, _S), jnp.int8),
        interpret=_INTERPRET,
    )(y, y)

    o = pl.pallas_call(
        _attn_kernel,
        grid=(_H, _S // _BQ),
        in_specs=[
            pl.BlockSpec((_BQ, _DH), lambda h, i: (i, h)),
            pl.BlockSpec((_S, _DH), lambda h, i: (0, 16 + h)),
            pl.BlockSpec((_S, _DH), lambda h, i: (0, 32 + h)),
            pl.BlockSpec((_BQ, _S), lambda h, i: (i, 0)),
        ],
        out_specs=pl.BlockSpec((_BQ, _DH), lambda h, i: (i, h)),
        out_shape=jax.ShapeDtypeStruct((_S, _H * _DH), jnp.float32),
        interpret=_INTERPRET,
    )(y, y, y, allow)

    h1 = pl.pallas_call(
        _matmul_res_kernel,
        grid=(_S // _BM, _D // _BN),
        in_specs=[
            pl.BlockSpec((_BM, _H * _DH), lambda i, j: (i, 0)),
            pl.BlockSpec((_H * _DH, _BN), lambda i, j: (0, j)),
            pl.BlockSpec((_BM, _BN), lambda i, j: (i, j)),
        ],
        out_specs=pl.BlockSpec((_BM, _BN), lambda i, j: (i, j)),
        out_shape=jax.ShapeDtypeStruct((_S, _D), jnp.float32),
        interpret=_INTERPRET,
    )(o, attn_wo, x)

    f = pl.pallas_call(
        _mlp_up_kernel,
        grid=(_S // _BM, _FF // _BN),
        in_specs=[
            pl.BlockSpec((_BM, _D), lambda i, j: (i, 0)),
            pl.BlockSpec((1, _D), lambda i, j: (0, 0)),
            pl.BlockSpec((_D, _BN), lambda i, j: (0, j)),
            pl.BlockSpec((_D, _BN), lambda i, j: (0, j)),
        ],
        out_specs=pl.BlockSpec((_BM, _BN), lambda i, j: (i, j)),
        out_shape=jax.ShapeDtypeStruct((_S, _FF), jnp.float32),
        scratch_shapes=[pltpu.VMEM((_BM, _D), jnp.float32)],
        interpret=_INTERPRET,
    )(h1, nw2, gate_w, up_w)

    out = pl.pallas_call(
        _matmul_res_kernel,
        grid=(_S // _BM, _D // _BN),
        in_specs=[
            pl.BlockSpec((_BM, _FF), lambda i, j: (i, 0)),
            pl.BlockSpec((_FF, _BN), lambda i, j: (0, j)),
            pl.BlockSpec((_BM, _BN), lambda i, j: (i, j)),
        ],
        out_specs=pl.BlockSpec((_BM, _BN), lambda i, j: (i, j)),
        out_shape=jax.ShapeDtypeStruct((_S, _D), jnp.float32),
        interpret=_INTERPRET,
    )(f, down_w, h1)

    return out[None]
